# baseline (speedup 1.0000x reference)
import functools

import jax
import jax.numpy as jnp
from jax import lax
from jax.experimental import pallas as pl
from jax.experimental.pallas import tpu as pltpu
from jax.experimental.pallas import tpu_sc as plsc

N = 10000
NP = 10240
D = 128
E = 320000
NC = 2
NS = 16
NB = 16
BN = NP // NB

N_PER_TILE = NP // NS
EPT = E // (NC * NS)
CHUNK = 125
NCHUNK = (E // (NC * NS)) // CHUNK
CHUNK_D = 1000
DEG_W = 1


def _deg_body(dst_hbm, zeros_hbm, ones_hbm, hist_hbm, idx_v, ones_v, acc_sh):
    c = lax.axis_index("c")
    s = lax.axis_index("s")
    sbase = s * (NP // NS)
    pltpu.sync_copy(zeros_hbm, acc_sh.at[pl.ds(sbase, NP // NS)])
    pltpu.sync_copy(ones_hbm, ones_v)
    plsc.subcore_barrier()

    ebase = (c * NS + s) * EPT
    pltpu.sync_copy(dst_hbm.at[pl.ds(ebase, EPT)], idx_v)
    pltpu.sync_copy(ones_v, acc_sh.at[idx_v], add=True)
    plsc.subcore_barrier()
    pltpu.sync_copy(
        acc_sh.at[pl.ds(sbase, NP // NS)],
        hist_hbm.at[pl.ds(c * NP + sbase, NP // NS)],
    )


def _scatter_body(
    hp_hbm, sd_hbm, zeros_hbm, acc_hbm,
    sd0_v, sd1_v, rows0_v, rows1_v, g0, g1, s0, s1, acc_sh
):
    c = lax.axis_index("c")
    s = lax.axis_index("s")
    base_n = s * N_PER_TILE
    pltpu.sync_copy(zeros_hbm, acc_sh.at[pl.ds(base_n, N_PER_TILE)])
    plsc.subcore_barrier()

    cbase = (c * NS + s) * NCHUNK

    def load_idx(i, sd_v):
        pltpu.sync_copy(sd_hbm.at[cbase + i], sd_v)

    def gather_start(sd_v, buf, sem):
        pltpu.async_copy(hp_hbm.at[sd_v.at[0]], buf, sem)

    def gather_wait(buf, sem):
        pltpu.make_async_copy(hp_hbm.at[sd0_v.at[0]], buf, sem).wait()

    def scat_start(sd_v, buf, sem):
        pltpu.async_copy(buf, acc_sh.at[sd_v.at[1]], sem, add=True)

    def scat_wait(sd_v, buf, sem):
        pltpu.make_async_copy(buf, acc_sh.at[sd_v.at[1]], sem).wait()

    load_idx(0, sd0_v)
    gather_start(sd0_v, rows0_v, g0)
    load_idx(1, sd1_v)
    gather_start(sd1_v, rows1_v, g1)

    def body(k, carry):
        i0 = 2 * k
        gather_wait(rows0_v, g0)
        scat_start(sd0_v, rows0_v, s0)
        gather_wait(rows1_v, g1)
        scat_start(sd1_v, rows1_v, s1)
        scat_wait(sd0_v, rows0_v, s0)
        load_idx(i0 + 2, sd0_v)
        gather_start(sd0_v, rows0_v, g0)
        scat_wait(sd1_v, rows1_v, s1)
        load_idx(i0 + 3, sd1_v)
        gather_start(sd1_v, rows1_v, g1)
        return carry

    lax.fori_loop(0, (NCHUNK - 2) // 2, body, 0)

    gather_wait(rows0_v, g0)
    scat_start(sd0_v, rows0_v, s0)
    gather_wait(rows1_v, g1)
    scat_start(sd1_v, rows1_v, s1)
    scat_wait(sd0_v, rows0_v, s0)
    scat_wait(sd1_v, rows1_v, s1)

    plsc.subcore_barrier()
    pltpu.sync_copy(
        acc_sh.at[pl.ds(base_n, N_PER_TILE)],
        acc_hbm.at[pl.ds(c * NP + base_n, N_PER_TILE)],
    )


@functools.lru_cache(maxsize=1)
def _sc_kernels():
    mesh = plsc.VectorSubcoreMesh(
        core_axis_name="c", subcore_axis_name="s", num_cores=NC, num_subcores=NS
    )
    deg = pl.kernel(
        _deg_body,
        out_type=jax.ShapeDtypeStruct((NC * NP * DEG_W,), jnp.float32),
        mesh=mesh,
        scratch_types=[
            pltpu.VMEM((EPT,), jnp.int32),
            pltpu.VMEM((EPT,), jnp.float32),
            pltpu.VMEM_SHARED((NP * DEG_W,), jnp.float32),
        ],
    )
    scatter = pl.kernel(
        _scatter_body,
        out_type=jax.ShapeDtypeStruct((NC * NP, D), jnp.float32),
        mesh=mesh,
        scratch_types=[
            pltpu.VMEM((2, CHUNK), jnp.int32),
            pltpu.VMEM((2, CHUNK), jnp.int32),
            pltpu.VMEM((CHUNK, D), jnp.float32),
            pltpu.VMEM((CHUNK, D), jnp.float32),
            pltpu.SemaphoreType.DMA,
            pltpu.SemaphoreType.DMA,
            pltpu.SemaphoreType.DMA,
            pltpu.SemaphoreType.DMA,
            pltpu.VMEM_SHARED((NP, D), jnp.float32),
        ],
    )
    return deg, scatter


def _tc1_body(hist0, hist1, x, w1, hp, dinv):
    deg = hist0[:, 0:1] + hist1[:, 0:1] + 1.0
    di = lax.rsqrt(deg)
    h = jnp.dot(x[...], w1[...], preferred_element_type=jnp.float32)
    hp[...] = h * di
    dinv[...] = di


def _tc1(hist2d, x, w1):
    return pl.pallas_call(
        _tc1_body,
        grid=(NB,),
        in_specs=[
            pl.BlockSpec((BN, DEG_W), lambda j: (j, 0)),
            pl.BlockSpec((BN, DEG_W), lambda j: (NB + j, 0)),
            pl.BlockSpec((BN, D), lambda j: (j, 0)),
            pl.BlockSpec((D, D), lambda j: (0, 0)),
        ],
        out_specs=[
            pl.BlockSpec((BN, D), lambda j: (j, 0)),
            pl.BlockSpec((BN, 1), lambda j: (j, 0)),
        ],
        out_shape=[
            jax.ShapeDtypeStruct((NP, D), jnp.float32),
            jax.ShapeDtypeStruct((NP, 1), jnp.float32),
        ],
    )(hist2d, hist2d, x, w1)


def _tc2_body(acc0, acc1, hp1, dinv, b1, w2, hp2):
    z = acc0[...] + acc1[...] + hp1[...]
    r = jnp.maximum(z * dinv[...] + b1[...], 0.0)
    h2 = jnp.dot(r, w2[...], preferred_element_type=jnp.float32)
    hp2[...] = h2 * dinv[...]


def _tc2(acc, hp1, dinv, b1, w2):
    return pl.pallas_call(
        _tc2_body,
        grid=(NB,),
        in_specs=[
            pl.BlockSpec((BN, D), lambda j: (j, 0)),
            pl.BlockSpec((BN, D), lambda j: (NB + j, 0)),
            pl.BlockSpec((BN, D), lambda j: (j, 0)),
            pl.BlockSpec((BN, 1), lambda j: (j, 0)),
            pl.BlockSpec((1, D), lambda j: (0, 0)),
            pl.BlockSpec((D, D), lambda j: (0, 0)),
        ],
        out_specs=pl.BlockSpec((BN, D), lambda j: (j, 0)),
        out_shape=jax.ShapeDtypeStruct((NP, D), jnp.float32),
    )(acc, acc, hp1, dinv, b1, w2)


def _tc3_body(acc0, acc1, hp2, dinv, b2, emb):
    z = acc0[...] + acc1[...] + hp2[...]
    emb[...] = z * dinv[...] + b2[...]


def _tc3(acc, hp2, dinv, b2):
    return pl.pallas_call(
        _tc3_body,
        grid=(NB,),
        in_specs=[
            pl.BlockSpec((BN, D), lambda j: (j, 0)),
            pl.BlockSpec((BN, D), lambda j: (NB + j, 0)),
            pl.BlockSpec((BN, D), lambda j: (j, 0)),
            pl.BlockSpec((BN, 1), lambda j: (j, 0)),
            pl.BlockSpec((1, D), lambda j: (0, 0)),
        ],
        out_specs=pl.BlockSpec((BN, D), lambda j: (j, 0)),
        out_shape=jax.ShapeDtypeStruct((NP, D), jnp.float32),
    )(acc, acc, hp2, dinv, b2)


def kernel(x, edge_index, W1, b1, W2, b2):
    src = edge_index[0].astype(jnp.int32)
    dst = edge_index[1].astype(jnp.int32)
    deg_kernel, scatter_kernel = _sc_kernels()
    zeros_deg = jnp.zeros((NP // NS,), jnp.float32)
    ones_deg = jnp.ones((EPT,), jnp.float32)
    hist = deg_kernel(dst, zeros_deg, ones_deg)
    hist2d = hist.reshape(NC * NP, DEG_W)

    xp = jnp.pad(x, ((0, NP - N), (0, 0)))
    zeros_rows = jnp.zeros((N_PER_TILE, D), jnp.float32)

    sd = jnp.stack([src.reshape(E // CHUNK, CHUNK), dst.reshape(E // CHUNK, CHUNK)], axis=1)
    hp1, dinv = _tc1(hist2d, xp, W1)
    acc1 = scatter_kernel(hp1, sd, zeros_rows)
    hp2 = _tc2(acc1, hp1, dinv, b1.reshape(1, D), W2)
    acc2 = scatter_kernel(hp2, sd, zeros_rows)
    return _tc3(acc2, hp2, dinv, b2.reshape(1, D))[:N]


# speedup vs baseline: 1.0066x; 1.0066x over previous
import functools

import jax
import jax.numpy as jnp
from jax import lax
from jax.experimental import pallas as pl
from jax.experimental.pallas import tpu as pltpu
from jax.experimental.pallas import tpu_sc as plsc

N = 10000
NP = 10240
D = 128
E = 320000
NC = 2
NS = 16
NB = 16
BN = NP // NB

N_PER_TILE = NP // NS
EPT = E // (NC * NS)
CHUNK = 80
NCHUNK = (E // (NC * NS)) // CHUNK
DEG_W = 1


def _deg_body(dst_hbm, zeros_hbm, ones_hbm, hist_hbm, idx_v, ones_v, acc_sh):
    c = lax.axis_index("c")
    s = lax.axis_index("s")
    sbase = s * (NP // NS)
    pltpu.sync_copy(zeros_hbm, acc_sh.at[pl.ds(sbase, NP // NS)])
    pltpu.sync_copy(ones_hbm, ones_v)
    plsc.subcore_barrier()

    ebase = (c * NS + s) * EPT
    pltpu.sync_copy(dst_hbm.at[pl.ds(ebase, EPT)], idx_v)
    pltpu.sync_copy(ones_v, acc_sh.at[idx_v], add=True)
    plsc.subcore_barrier()
    pltpu.sync_copy(
        acc_sh.at[pl.ds(sbase, NP // NS)],
        hist_hbm.at[pl.ds(c * NP + sbase, NP // NS)],
    )


def _scatter_body(
    hp_hbm, sd_hbm, zeros_hbm, acc_hbm,
    sd0_v, sd1_v, sd2_v, rows0_v, rows1_v, rows2_v, g0, g1, g2, s0, s1, s2, acc_sh
):
    c = lax.axis_index("c")
    s = lax.axis_index("s")
    base_n = s * N_PER_TILE
    pltpu.sync_copy(zeros_hbm, acc_sh.at[pl.ds(base_n, N_PER_TILE)])
    plsc.subcore_barrier()

    cbase = (c * NS + s) * NCHUNK

    def load_idx(i, sd_v):
        pltpu.sync_copy(sd_hbm.at[cbase + i], sd_v)

    def gather_start(sd_v, buf, sem):
        pltpu.async_copy(hp_hbm.at[sd_v.at[0]], buf, sem)

    def gather_wait(buf, sem):
        pltpu.make_async_copy(hp_hbm.at[sd0_v.at[0]], buf, sem).wait()

    def scat_start(sd_v, buf, sem):
        pltpu.async_copy(buf, acc_sh.at[sd_v.at[1]], sem, add=True)

    def scat_wait(sd_v, buf, sem):
        pltpu.make_async_copy(buf, acc_sh.at[sd_v.at[1]], sem).wait()

    load_idx(0, sd0_v)
    gather_start(sd0_v, rows0_v, g0)
    load_idx(1, sd1_v)
    gather_start(sd1_v, rows1_v, g1)
    load_idx(2, sd2_v)
    gather_start(sd2_v, rows2_v, g2)

    def body(k, carry):
        i0 = 3 * k
        gather_wait(rows0_v, g0)
        scat_start(sd0_v, rows0_v, s0)
        gather_wait(rows1_v, g1)
        scat_start(sd1_v, rows1_v, s1)
        gather_wait(rows2_v, g2)
        scat_start(sd2_v, rows2_v, s2)
        scat_wait(sd0_v, rows0_v, s0)
        load_idx(i0 + 3, sd0_v)
        gather_start(sd0_v, rows0_v, g0)
        scat_wait(sd1_v, rows1_v, s1)
        load_idx(i0 + 4, sd1_v)
        gather_start(sd1_v, rows1_v, g1)
        scat_wait(sd2_v, rows2_v, s2)
        load_idx(i0 + 5, sd2_v)
        gather_start(sd2_v, rows2_v, g2)
        return carry

    lax.fori_loop(0, (NCHUNK - 5) // 3, body, 0)

    gather_wait(rows0_v, g0)
    scat_start(sd0_v, rows0_v, s0)
    scat_wait(sd0_v, rows0_v, s0)
    load_idx(NCHUNK - 2, sd0_v)
    gather_start(sd0_v, rows0_v, g0)
    gather_wait(rows1_v, g1)
    scat_start(sd1_v, rows1_v, s1)
    scat_wait(sd1_v, rows1_v, s1)
    load_idx(NCHUNK - 1, sd1_v)
    gather_start(sd1_v, rows1_v, g1)
    gather_wait(rows2_v, g2)
    scat_start(sd2_v, rows2_v, s2)
    gather_wait(rows0_v, g0)
    scat_start(sd0_v, rows0_v, s0)
    gather_wait(rows1_v, g1)
    scat_start(sd1_v, rows1_v, s1)
    scat_wait(sd2_v, rows2_v, s2)
    scat_wait(sd0_v, rows0_v, s0)
    scat_wait(sd1_v, rows1_v, s1)

    plsc.subcore_barrier()
    pltpu.sync_copy(
        acc_sh.at[pl.ds(base_n, N_PER_TILE)],
        acc_hbm.at[pl.ds(c * NP + base_n, N_PER_TILE)],
    )


@functools.lru_cache(maxsize=1)
def _sc_kernels():
    mesh = plsc.VectorSubcoreMesh(
        core_axis_name="c", subcore_axis_name="s", num_cores=NC, num_subcores=NS
    )
    deg = pl.kernel(
        _deg_body,
        out_type=jax.ShapeDtypeStruct((NC * NP * DEG_W,), jnp.float32),
        mesh=mesh,
        scratch_types=[
            pltpu.VMEM((EPT,), jnp.int32),
            pltpu.VMEM((EPT,), jnp.float32),
            pltpu.VMEM_SHARED((NP * DEG_W,), jnp.float32),
        ],
    )
    scatter = pl.kernel(
        _scatter_body,
        out_type=jax.ShapeDtypeStruct((NC * NP, D), jnp.float32),
        mesh=mesh,
        scratch_types=[
            pltpu.VMEM((2, CHUNK), jnp.int32),
            pltpu.VMEM((2, CHUNK), jnp.int32),
            pltpu.VMEM((2, CHUNK), jnp.int32),
            pltpu.VMEM((CHUNK, D), jnp.float32),
            pltpu.VMEM((CHUNK, D), jnp.float32),
            pltpu.VMEM((CHUNK, D), jnp.float32),
            pltpu.SemaphoreType.DMA,
            pltpu.SemaphoreType.DMA,
            pltpu.SemaphoreType.DMA,
            pltpu.SemaphoreType.DMA,
            pltpu.SemaphoreType.DMA,
            pltpu.SemaphoreType.DMA,
            pltpu.VMEM_SHARED((NP, D), jnp.float32),
        ],
    )
    return deg, scatter


def _tc1_body(hist0, hist1, x, w1, hp, dinv):
    deg = hist0[:, 0:1] + hist1[:, 0:1] + 1.0
    di = lax.rsqrt(deg)
    h = jnp.dot(x[...], w1[...], preferred_element_type=jnp.float32)
    hp[...] = h * di
    dinv[...] = di


def _tc1(hist2d, x, w1):
    return pl.pallas_call(
        _tc1_body,
        grid=(NB,),
        in_specs=[
            pl.BlockSpec((BN, DEG_W), lambda j: (j, 0)),
            pl.BlockSpec((BN, DEG_W), lambda j: (NB + j, 0)),
            pl.BlockSpec((BN, D), lambda j: (j, 0)),
            pl.BlockSpec((D, D), lambda j: (0, 0)),
        ],
        out_specs=[
            pl.BlockSpec((BN, D), lambda j: (j, 0)),
            pl.BlockSpec((BN, 1), lambda j: (j, 0)),
        ],
        out_shape=[
            jax.ShapeDtypeStruct((NP, D), jnp.float32),
            jax.ShapeDtypeStruct((NP, 1), jnp.float32),
        ],
    )(hist2d, hist2d, x, w1)


def _tc2_body(acc0, acc1, hp1, dinv, b1, w2, hp2):
    z = acc0[...] + acc1[...] + hp1[...]
    r = jnp.maximum(z * dinv[...] + b1[...], 0.0)
    h2 = jnp.dot(r, w2[...], preferred_element_type=jnp.float32)
    hp2[...] = h2 * dinv[...]


def _tc2(acc, hp1, dinv, b1, w2):
    return pl.pallas_call(
        _tc2_body,
        grid=(NB,),
        in_specs=[
            pl.BlockSpec((BN, D), lambda j: (j, 0)),
            pl.BlockSpec((BN, D), lambda j: (NB + j, 0)),
            pl.BlockSpec((BN, D), lambda j: (j, 0)),
            pl.BlockSpec((BN, 1), lambda j: (j, 0)),
            pl.BlockSpec((1, D), lambda j: (0, 0)),
            pl.BlockSpec((D, D), lambda j: (0, 0)),
        ],
        out_specs=pl.BlockSpec((BN, D), lambda j: (j, 0)),
        out_shape=jax.ShapeDtypeStruct((NP, D), jnp.float32),
    )(acc, acc, hp1, dinv, b1, w2)


def _tc3_body(acc0, acc1, hp2, dinv, b2, emb):
    z = acc0[...] + acc1[...] + hp2[...]
    emb[...] = z * dinv[...] + b2[...]


def _tc3(acc, hp2, dinv, b2):
    return pl.pallas_call(
        _tc3_body,
        grid=(NB,),
        in_specs=[
            pl.BlockSpec((BN, D), lambda j: (j, 0)),
            pl.BlockSpec((BN, D), lambda j: (NB + j, 0)),
            pl.BlockSpec((BN, D), lambda j: (j, 0)),
            pl.BlockSpec((BN, 1), lambda j: (j, 0)),
            pl.BlockSpec((1, D), lambda j: (0, 0)),
        ],
        out_specs=pl.BlockSpec((BN, D), lambda j: (j, 0)),
        out_shape=jax.ShapeDtypeStruct((NP, D), jnp.float32),
    )(acc, acc, hp2, dinv, b2)


def kernel(x, edge_index, W1, b1, W2, b2):
    src = edge_index[0].astype(jnp.int32)
    dst = edge_index[1].astype(jnp.int32)
    deg_kernel, scatter_kernel = _sc_kernels()
    zeros_deg = jnp.zeros((NP // NS,), jnp.float32)
    ones_deg = jnp.ones((EPT,), jnp.float32)
    hist = deg_kernel(dst, zeros_deg, ones_deg)
    hist2d = hist.reshape(NC * NP, DEG_W)

    xp = jnp.pad(x, ((0, NP - N), (0, 0)))
    zeros_rows = jnp.zeros((N_PER_TILE, D), jnp.float32)

    sd = jnp.stack([src.reshape(E // CHUNK, CHUNK), dst.reshape(E // CHUNK, CHUNK)], axis=1)
    hp1, dinv = _tc1(hist2d, xp, W1)
    acc1 = scatter_kernel(hp1, sd, zeros_rows)
    hp2 = _tc2(acc1, hp1, dinv, b1.reshape(1, D), W2)
    acc2 = scatter_kernel(hp2, sd, zeros_rows)
    return _tc3(acc2, hp2, dinv, b2.reshape(1, D))[:N]
